# Initial kernel scaffold; baseline (speedup 1.0000x reference)
import jax
import jax.numpy as jnp
from jax.experimental import pallas as pl


def kernel(x, edge_index, W1, b1, W2, b2):
    raise NotImplementedError("write your pallas kernel here")


# speedup vs baseline: 5.3890x; 5.3890x over previous
import functools

import jax
import jax.numpy as jnp
from jax import lax
from jax.experimental import pallas as pl
from jax.experimental.pallas import tpu as pltpu
from jax.experimental.pallas import tpu_sc as plsc

N = 10000
E = 320000
D = 128
EPS = 0.0

NC = 2
NS = 16
NW = NC * NS
EPW = E // NW
K = 80
NCHUNK = EPW // K
ZB = 80
NB = N // ZB
BPT = -(-NB // NS)


def _sc_agg_kernel(x_hbm, src_hbm, dst_hbm, out_hbm,
                   acc, sidx, didx, rows, zbuf, sem):
    cid = lax.axis_index("c")
    tid = lax.axis_index("s")
    wid = cid * NS + tid

    def zero_body(i, _):
        r = i // (D // 16)
        c = (i % (D // 16)) * 16
        zbuf[r, pl.ds(c, 16)] = jnp.zeros((16,), jnp.float32)
        return 0

    lax.fori_loop(0, ZB * (D // 16), zero_body, 0)

    def zcopy_body(j, _):
        b = j * NS + tid

        @pl.when(b < NB)
        def _():
            pltpu.sync_copy(zbuf, acc.at[pl.ds(b * ZB, ZB)])

        return 0

    lax.fori_loop(0, BPT, zcopy_body, 0)
    plsc.subcore_barrier()

    ebase = wid * EPW

    def edge_body(i, _):
        off = ebase + i * K
        pltpu.sync_copy(src_hbm.at[pl.ds(off, K)], sidx)
        pltpu.sync_copy(dst_hbm.at[pl.ds(off, K)], didx)
        pltpu.async_copy(x_hbm.at[sidx], rows, sem).wait()
        pltpu.sync_copy(rows, acc.at[didx], add=True)
        return 0

    lax.fori_loop(0, NCHUNK, edge_body, 0)
    plsc.subcore_barrier()

    def out_body(j, _):
        b = j * NS + tid

        @pl.when(b < NB)
        def _():
            r = b * ZB
            pltpu.sync_copy(acc.at[pl.ds(r, ZB)], zbuf)
            pltpu.sync_copy(zbuf, out_hbm.at[cid, pl.ds(r, ZB)])

        return 0

    lax.fori_loop(0, BPT, out_body, 0)


def _sc_aggregate(x, src, dst):
    mesh = plsc.VectorSubcoreMesh(
        core_axis_name="c", subcore_axis_name="s",
        num_cores=NC, num_subcores=NS)
    return pl.kernel(
        _sc_agg_kernel,
        out_type=jax.ShapeDtypeStruct((NC, N, D), jnp.float32),
        mesh=mesh,
        scratch_types=[
            pltpu.VMEM_SHARED((N, D), jnp.float32),
            pltpu.VMEM((K,), jnp.int32),
            pltpu.VMEM((K,), jnp.int32),
            pltpu.VMEM((K, D), jnp.float32),
            pltpu.VMEM((ZB, D), jnp.float32),
            pltpu.SemaphoreType.DMA,
        ],
    )(x, src, dst)


def _tc_mlp_kernel(x_ref, p0_ref, p1_ref, w1_ref, b1_ref, w2_ref, b2_ref,
                   out_ref):
    s = (1.0 + EPS) * x_ref[...] + p0_ref[...] + p1_ref[...]
    h = jnp.dot(s, w1_ref[...], preferred_element_type=jnp.float32)
    h = jnp.maximum(h + b1_ref[...], 0.0)
    z = jnp.dot(h, w2_ref[...], preferred_element_type=jnp.float32)
    out_ref[...] = z + b2_ref[...]


def _tc_mlp(x, p0, p1, W1, b1, W2, b2):
    BT = 1000
    grid = (N // BT,)
    row_spec = pl.BlockSpec((BT, D), lambda i: (i, 0))
    full = pl.BlockSpec((D, D), lambda i: (0, 0))
    bias = pl.BlockSpec((1, D), lambda i: (0, 0))
    return pl.pallas_call(
        _tc_mlp_kernel,
        grid=grid,
        in_specs=[row_spec, row_spec, row_spec, full, bias, full, bias],
        out_specs=row_spec,
        out_shape=jax.ShapeDtypeStruct((N, D), jnp.float32),
    )(x, p0, p1, W1, b1.reshape(1, D), W2, b2.reshape(1, D))


@jax.jit
def kernel(x, edge_index, W1, b1, W2, b2):
    src = edge_index[0]
    dst = edge_index[1]
    partials = _sc_aggregate(x, src, dst)
    return _tc_mlp(x, partials[0], partials[1], W1, b1, W2, b2)


# speedup vs baseline: 9.2240x; 1.7116x over previous
import functools

import jax
import jax.numpy as jnp
from jax import lax
from jax.experimental import pallas as pl
from jax.experimental.pallas import tpu as pltpu
from jax.experimental.pallas import tpu_sc as plsc

N = 10000
E = 320000
D = 128
EPS = 0.0

NC = 2
NS = 16
NW = NC * NS
EPW = E // NW
K = 80
NCHUNK = EPW // K
ZB = 80
NB = N // ZB
BPT = -(-NB // NS)


def _sc_agg_kernel(x_hbm, src_hbm, dst_hbm, out_hbm,
                   acc, sidx, didx, rows, gsem, ssem):
    cid = lax.axis_index("c")
    tid = lax.axis_index("s")
    wid = cid * NS + tid

    zbuf = rows.at[0]

    def zero_body(i, _):
        r = i // (D // 16)
        c = (i % (D // 16)) * 16
        zbuf[r, pl.ds(c, 16)] = jnp.zeros((16,), jnp.float32)
        return 0

    lax.fori_loop(0, ZB * (D // 16), zero_body, 0)

    def zcopy_body(j, _):
        b = j * NS + tid

        @pl.when(b < NB)
        def _():
            pltpu.sync_copy(zbuf, acc.at[pl.ds(b * ZB, ZB)])

        return 0

    lax.fori_loop(0, BPT, zcopy_body, 0)
    plsc.subcore_barrier()

    pltpu.sync_copy(src_hbm.at[wid], sidx)
    pltpu.sync_copy(dst_hbm.at[wid], didx)

    def gstart(c, b):
        pltpu.async_copy(x_hbm.at[sidx.at[pl.ds(c * K, K)]], rows.at[b], gsem)

    def sstart(c, b):
        pltpu.async_copy(rows.at[b], acc.at[didx.at[c]], ssem, add=True)

    def gwait():
        pltpu.make_async_copy(x_hbm.at[pl.ds(0, K)], rows.at[0], gsem).wait()

    def swait():
        pltpu.make_async_copy(x_hbm.at[pl.ds(0, K)], rows.at[0], ssem).wait()

    gstart(0, 0)

    def edge_body(i, _):
        b = i % 2
        gwait()
        pl.when(i >= 1)(swait)
        pl.when(i + 1 < NCHUNK)(lambda: gstart(i + 1, 1 - b))
        sstart(i, b)
        return 0

    lax.fori_loop(0, NCHUNK, edge_body, 0)
    swait()
    plsc.subcore_barrier()

    obuf = rows.at[1]

    def out_body(j, _):
        b = j * NS + tid

        @pl.when(b < NB)
        def _():
            r = b * ZB
            pltpu.sync_copy(acc.at[pl.ds(r, ZB)], obuf)
            pltpu.sync_copy(obuf, out_hbm.at[cid, pl.ds(r, ZB)])

        return 0

    lax.fori_loop(0, BPT, out_body, 0)


def _sc_aggregate(x, src, dst):
    mesh = plsc.VectorSubcoreMesh(
        core_axis_name="c", subcore_axis_name="s",
        num_cores=NC, num_subcores=NS)
    return pl.kernel(
        _sc_agg_kernel,
        out_type=jax.ShapeDtypeStruct((NC, N, D), jnp.float32),
        mesh=mesh,
        scratch_types=[
            pltpu.VMEM_SHARED((N, D), jnp.float32),
            pltpu.VMEM((EPW,), jnp.int32),
            pltpu.VMEM((NCHUNK, K), jnp.int32),
            pltpu.VMEM((2, K, D), jnp.float32),
            pltpu.SemaphoreType.DMA,
            pltpu.SemaphoreType.DMA,
        ],
    )(x, src, dst)


def _tc_mlp_kernel(x_ref, p0_ref, p1_ref, w1_ref, b1_ref, w2_ref, b2_ref,
                   out_ref):
    s = (1.0 + EPS) * x_ref[...] + p0_ref[...] + p1_ref[...]
    h = jnp.dot(s, w1_ref[...], preferred_element_type=jnp.float32)
    h = jnp.maximum(h + b1_ref[...], 0.0)
    z = jnp.dot(h, w2_ref[...], preferred_element_type=jnp.float32)
    out_ref[...] = z + b2_ref[...]


def _tc_mlp(x, p0, p1, W1, b1, W2, b2):
    BT = 1000
    grid = (N // BT,)
    row_spec = pl.BlockSpec((BT, D), lambda i: (i, 0))
    full = pl.BlockSpec((D, D), lambda i: (0, 0))
    bias = pl.BlockSpec((1, D), lambda i: (0, 0))
    return pl.pallas_call(
        _tc_mlp_kernel,
        grid=grid,
        in_specs=[row_spec, row_spec, row_spec, full, bias, full, bias],
        out_specs=row_spec,
        out_shape=jax.ShapeDtypeStruct((N, D), jnp.float32),
    )(x, p0, p1, W1, b1.reshape(1, D), W2, b2.reshape(1, D))


@jax.jit
def kernel(x, edge_index, W1, b1, W2, b2):
    src = edge_index[0].reshape(NW, EPW)
    dst = edge_index[1].reshape(NW, NCHUNK, K)
    partials = _sc_aggregate(x, src, dst)
    return _tc_mlp(x, partials[0], partials[1], W1, b1, W2, b2)
